# Initial kernel scaffold; baseline (speedup 1.0000x reference)
import jax
import jax.numpy as jnp
from jax.experimental import pallas as pl


def kernel(points, W, b):
    raise NotImplementedError("write your pallas kernel here")


# speedup vs baseline: 5.4448x; 5.4448x over previous
import jax
import jax.numpy as jnp
from jax.experimental import pallas as pl

B = 4
N = 4096
K = 8
C = 8
TQ = 256


def _knn_feature_body(q_ref, kt_ref, w_ref, b_ref, out_ref):
    q = q_ref[0]
    kt = kt_ref[0]
    w = w_ref[...]
    bias = b_ref[...]

    sq_k = jnp.sum(kt * kt, axis=0, keepdims=True)
    sq_q = jnp.sum(q * q, axis=1, keepdims=True)
    inner = jax.lax.dot_general(
        q, kt, (((1,), (0,)), ((), ())),
        preferred_element_type=jnp.float32)
    d = sq_q - 2.0 * inner + sq_k

    f_keys = jax.lax.dot_general(
        kt, w, (((0,), (1,)), ((), ())),
        preferred_element_type=jnp.float32,
        precision=jax.lax.Precision.HIGHEST) + bias
    f_q = jax.lax.dot_general(
        q, w, (((1,), (1,)), ((), ())),
        preferred_element_type=jnp.float32,
        precision=jax.lax.Precision.HIGHEST) + bias

    iota = jax.lax.broadcasted_iota(jnp.int32, (TQ, N), 1)
    big = jnp.float32(3.0e38)
    pieces = []
    for _ in range(K):
        m = jnp.min(d, axis=1, keepdims=True)
        eq = d == m
        first = jnp.min(jnp.where(eq, iota, N), axis=1, keepdims=True)
        onehot = iota == first
        g = jax.lax.dot_general(
            onehot.astype(jnp.float32), f_keys,
            (((1,), (0,)), ((), ())),
            preferred_element_type=jnp.float32,
            precision=jax.lax.Precision.HIGHEST)
        pieces.append(g - f_q)
        pieces.append(f_q)
        d = jnp.where(onehot, big, d)
    out_ref[0] = jnp.concatenate(pieces, axis=1)


def kernel(points, W, b):
    points_t = jnp.transpose(points, (0, 2, 1))
    bias = b.reshape(1, C)
    grid = (B, N // TQ)
    out = pl.pallas_call(
        _knn_feature_body,
        grid=grid,
        in_specs=[
            pl.BlockSpec((1, TQ, 3), lambda i, j: (i, j, 0)),
            pl.BlockSpec((1, 3, N), lambda i, j: (i, 0, 0)),
            pl.BlockSpec((C, 3), lambda i, j: (0, 0)),
            pl.BlockSpec((1, C), lambda i, j: (0, 0)),
        ],
        out_specs=pl.BlockSpec((1, TQ, K * 2 * C), lambda i, j: (i, j, 0)),
        out_shape=jax.ShapeDtypeStruct((B, N, K * 2 * C), jnp.float32),
    )(points, points_t, W, bias)
    return jnp.transpose(out.reshape(B, N, K, 2 * C), (0, 3, 1, 2))


# speedup vs baseline: 24.8553x; 4.5650x over previous
import functools

import jax
import jax.numpy as jnp
from jax import lax
from jax.experimental import pallas as pl
from jax.experimental.pallas import tpu as pltpu
from jax.experimental.pallas import tpu_sc as plsc

B = 4
N = 4096
K = 8
C = 8
TQ = 256

NC = 2
NS = 16
NW = NC * NS
QW = (B * N) // NW
RW = QW * K
GCH = 128
NCH = RW // GCH


def _knn_idx_body(q_ref, kt_ref, w_ref, b_ref, fdup_ref, idx_ref):
    q = q_ref[0]
    kt = kt_ref[0]
    w = w_ref[...]
    bias = b_ref[...]

    sq_k = jnp.sum(kt * kt, axis=0, keepdims=True)
    sq_q = jnp.sum(q * q, axis=1, keepdims=True)
    inner = jax.lax.dot_general(
        q, kt, (((1,), (0,)), ((), ())),
        preferred_element_type=jnp.float32)
    d = sq_q - 2.0 * inner + sq_k

    f_q = jax.lax.dot_general(
        q, w, (((1,), (1,)), ((), ())),
        preferred_element_type=jnp.float32,
        precision=jax.lax.Precision.HIGHEST) + bias
    fdup_ref[...] = jnp.concatenate([f_q, f_q], axis=1)

    iota = jax.lax.broadcasted_iota(jnp.int32, (TQ, N), 1)
    big = jnp.float32(3.0e38)
    cols = []
    for _ in range(K):
        m = jnp.min(d, axis=1, keepdims=True)
        eq = d == m
        first = jnp.min(jnp.where(eq, iota, N), axis=1, keepdims=True)
        cols.append(first)
        d = jnp.where(iota == first, big, d)
    base = pl.program_id(0) * N
    idx_ref[...] = jnp.concatenate(cols, axis=1) + base


_SC_MESH = plsc.VectorSubcoreMesh(core_axis_name="c", subcore_axis_name="s")


@functools.partial(
    pl.kernel,
    out_type=jax.ShapeDtypeStruct((B * N * K, 2 * C), jnp.float32),
    mesh=_SC_MESH,
    scratch_types=[
        pltpu.VMEM((NCH, GCH), jnp.int32),
        pltpu.VMEM((RW, 2 * C), jnp.float32),
        pltpu.VMEM((QW, 2 * C), jnp.float32),
        pltpu.SemaphoreType.DMA,
    ],
    compiler_params=pltpu.CompilerParams(use_tc_tiling_on_sc=False),
)
def _sc_gather(fdup_hbm, idxg_hbm, out_hbm, idx_v, rows_v, fq_v, gsem):
    wid = lax.axis_index("s") * NC + lax.axis_index("c")
    qbase = wid * QW
    pltpu.sync_copy(idxg_hbm.at[pl.ds(wid * NCH, NCH)], idx_v)
    pltpu.sync_copy(fdup_hbm.at[pl.ds(qbase, QW)], fq_v)
    copies = []
    for c in range(NCH):
        copies.append(
            pltpu.async_copy(
                fdup_hbm.at[idx_v.at[c]],
                rows_v.at[pl.ds(c * GCH, GCH)],
                gsem))
    for cp in copies:
        cp.wait()

    half = lax.iota(jnp.int32, 2 * C) < C

    def body(qi, _):
        fq = fq_v[qi]
        for j in range(K):
            r = qi * K + j
            v = rows_v[r]
            rows_v[r] = jnp.where(half, v - fq, fq)
        return _

    lax.fori_loop(0, QW, body, None)
    pltpu.sync_copy(rows_v, out_hbm.at[pl.ds(wid * RW, RW)])


def kernel(points, W, b):
    points_t = jnp.transpose(points, (0, 2, 1))
    bias = b.reshape(1, C)
    grid = (B, N // TQ)
    fdup, idxg = pl.pallas_call(
        _knn_idx_body,
        grid=grid,
        in_specs=[
            pl.BlockSpec((1, TQ, 3), lambda i, j: (i, j, 0)),
            pl.BlockSpec((1, 3, N), lambda i, j: (i, 0, 0)),
            pl.BlockSpec((C, 3), lambda i, j: (0, 0)),
            pl.BlockSpec((1, C), lambda i, j: (0, 0)),
        ],
        out_specs=[
            pl.BlockSpec((TQ, 2 * C), lambda i, j: (i * (N // TQ) + j, 0)),
            pl.BlockSpec((TQ, K), lambda i, j: (i * (N // TQ) + j, 0)),
        ],
        out_shape=[
            jax.ShapeDtypeStruct((B * N, 2 * C), jnp.float32),
            jax.ShapeDtypeStruct((B * N, K), jnp.int32),
        ],
    )(points, points_t, W, bias)
    out = _sc_gather(fdup, idxg.reshape(B * N * K // GCH, GCH))
    return jnp.transpose(out.reshape(B, N, K, 2 * C), (0, 3, 1, 2))


# speedup vs baseline: 46.4126x; 1.8673x over previous
import functools

import jax
import jax.numpy as jnp
from jax import lax
from jax.experimental import pallas as pl
from jax.experimental.pallas import tpu as pltpu
from jax.experimental.pallas import tpu_sc as plsc

B = 4
N = 4096
K = 8
C = 8
TQ = 256

NC = 2
NS = 16
NW = NC * NS
QW = N // NW
RW = QW * K
GCH = 128
NCH = RW // GCH


def _knn_idx_body(q_ref, kt_ref, w_ref, b_ref, fdup_ref, idx_ref):
    q = q_ref[...]
    kt = kt_ref[...]
    w = w_ref[...]
    bias = b_ref[...]

    sq_k = jnp.sum(kt * kt, axis=0, keepdims=True)
    sq_q = jnp.sum(q * q, axis=1, keepdims=True)
    inner = jax.lax.dot_general(
        q, kt, (((1,), (0,)), ((), ())),
        preferred_element_type=jnp.float32)
    d = sq_q - 2.0 * inner + sq_k

    f_q = jax.lax.dot_general(
        q, w, (((1,), (1,)), ((), ())),
        preferred_element_type=jnp.float32,
        precision=jax.lax.Precision.HIGHEST) + bias
    fdup_ref[...] = jnp.concatenate([f_q, f_q], axis=1)

    keys = (jax.lax.bitcast_convert_type(d + 1.0, jnp.uint32)
            & jnp.uint32(0xFFFFFFE0))
    big = jnp.float32(3.0e38)
    planes = [jnp.full((TQ, 128), big, jnp.float32) for _ in range(4)]
    for c in range(32):
        cv = jax.lax.bitcast_convert_type(
            keys[:, c * 128:(c + 1) * 128] | jnp.uint32(c), jnp.float32)
        for l in range(4):
            lo = jnp.minimum(planes[l], cv)
            cv = jnp.maximum(planes[l], cv)
            planes[l] = lo

    lane = jax.lax.broadcasted_iota(
        jnp.int32, (TQ, 128), 1).astype(jnp.float32)
    cand = jnp.concatenate(planes, axis=1)
    oidx = jnp.concatenate(
        [(jax.lax.bitcast_convert_type(p, jnp.uint32)
          & jnp.uint32(31)).astype(jnp.float32) * 128.0 + lane
         for p in planes], axis=1)
    nf = jnp.float32(N)
    cols = []
    for j in range(K):
        m = jnp.min(cand, axis=1, keepdims=True)
        first = jnp.min(jnp.where(cand == m, oidx, nf), axis=1,
                        keepdims=True)
        cols.append(first)
        if j < K - 1:
            cand = jnp.where(oidx == first, big, cand)
    idx_ref[...] = jnp.concatenate(cols, axis=1).astype(jnp.int32)


_SC_MESH = plsc.VectorSubcoreMesh(core_axis_name="c", subcore_axis_name="s")


@functools.partial(
    pl.kernel,
    out_type=jax.ShapeDtypeStruct((N * K, 2 * C), jnp.float32),
    mesh=_SC_MESH,
    scratch_types=[
        pltpu.VMEM((NCH, GCH), jnp.int32),
        pltpu.VMEM((RW, 2 * C), jnp.float32),
        pltpu.VMEM((QW, 2 * C), jnp.float32),
        pltpu.SemaphoreType.DMA,
    ],
    compiler_params=pltpu.CompilerParams(use_tc_tiling_on_sc=False),
)
def _sc_gather(fdup_hbm, idxg_hbm, out_hbm, idx_v, rows_v, fq_v, gsem):
    wid = lax.axis_index("s") * NC + lax.axis_index("c")
    qbase = wid * QW
    pltpu.sync_copy(idxg_hbm.at[pl.ds(wid * NCH, NCH)], idx_v)
    pltpu.sync_copy(fdup_hbm.at[pl.ds(qbase, QW)], fq_v)
    copies = []
    for c in range(NCH):
        copies.append(
            pltpu.async_copy(
                fdup_hbm.at[idx_v.at[c]],
                rows_v.at[pl.ds(c * GCH, GCH)],
                gsem))
    for cp in copies:
        cp.wait()

    half = lax.iota(jnp.int32, 2 * C) < C

    def body(qi, _):
        fq = fq_v[qi]
        for j in range(K):
            r = qi * K + j
            v = rows_v[r]
            rows_v[r] = jnp.where(half, v - fq, fq)
        return _

    lax.fori_loop(0, QW, body, None)
    pltpu.sync_copy(rows_v, out_hbm.at[pl.ds(wid * RW, RW)])


def kernel(points, W, b):
    points_t = jnp.transpose(points, (0, 2, 1))
    bias = b.reshape(1, C)
    tc = pl.pallas_call(
        _knn_idx_body,
        grid=(N // TQ,),
        in_specs=[
            pl.BlockSpec((TQ, 3), lambda j: (j, 0)),
            pl.BlockSpec((3, N), lambda j: (0, 0)),
            pl.BlockSpec((C, 3), lambda j: (0, 0)),
            pl.BlockSpec((1, C), lambda j: (0, 0)),
        ],
        out_specs=[
            pl.BlockSpec((TQ, 2 * C), lambda j: (j, 0)),
            pl.BlockSpec((TQ, K), lambda j: (j, 0)),
        ],
        out_shape=[
            jax.ShapeDtypeStruct((N, 2 * C), jnp.float32),
            jax.ShapeDtypeStruct((N, K), jnp.int32),
        ],
    )
    outs = []
    for bi in range(B):
        fdup, idxg = tc(points[bi], points_t[bi], W, bias)
        outs.append(_sc_gather(fdup, idxg.reshape(N * K // GCH, GCH)))
    out = jnp.stack(outs)
    return jnp.transpose(out.reshape(B, N, K, 2 * C), (0, 3, 1, 2))


# speedup vs baseline: 50.9074x; 1.0968x over previous
import functools

import jax
import jax.numpy as jnp
from jax import lax
from jax.experimental import pallas as pl
from jax.experimental.pallas import tpu as pltpu
from jax.experimental.pallas import tpu_sc as plsc

B = 4
N = 4096
K = 8
C = 8
TQ = 256

NC = 2
NS = 16
NW = NC * NS
QW = (B * N) // NW
RW = QW * K
GCH = 128
NCH = RW // GCH


def _knn_idx_body(q_ref, kt_ref, w_ref, b_ref, fdup_ref, idx_ref, sqk_ref):
    q = q_ref[0]
    kt = kt_ref[0]
    w = w_ref[...]
    bias = b_ref[...]

    @pl.when(pl.program_id(1) == 0)
    def _():
        sqk_ref[...] = jnp.sum(kt * kt, axis=0, keepdims=True)

    sq_k = sqk_ref[...]
    sq_q = jnp.sum(q * q, axis=1, keepdims=True)
    inner = jax.lax.dot_general(
        q, kt, (((1,), (0,)), ((), ())),
        preferred_element_type=jnp.float32)
    d = sq_q - 2.0 * inner + sq_k

    f_q = jax.lax.dot_general(
        q, w, (((1,), (1,)), ((), ())),
        preferred_element_type=jnp.float32,
        precision=jax.lax.Precision.HIGHEST) + bias
    fdup_ref[...] = jnp.concatenate([f_q, f_q], axis=1)

    keys = (jax.lax.bitcast_convert_type(
        jnp.maximum(d, jnp.float32(1e-30)), jnp.uint32)
            & jnp.uint32(0xFFFFFFE0))
    big = jnp.float32(3.0e38)
    planes = [jnp.full((TQ, 128), big, jnp.float32) for _ in range(4)]
    for c in range(32):
        cv = jax.lax.bitcast_convert_type(
            keys[:, c * 128:(c + 1) * 128] | jnp.uint32(c), jnp.float32)
        for l in range(4):
            lo = jnp.minimum(planes[l], cv)
            cv = jnp.maximum(planes[l], cv)
            planes[l] = lo

    lane = jax.lax.broadcasted_iota(
        jnp.int32, (TQ, 128), 1).astype(jnp.float32)
    cand = jnp.concatenate(planes, axis=1)
    oidx = jnp.concatenate(
        [(jax.lax.bitcast_convert_type(p, jnp.uint32)
          & jnp.uint32(31)).astype(jnp.float32) * 128.0 + lane
         for p in planes], axis=1)
    nf = jnp.float32(N)
    cols = []
    for j in range(K):
        m = jnp.min(cand, axis=1, keepdims=True)
        first = jnp.min(jnp.where(cand == m, oidx, nf), axis=1,
                        keepdims=True)
        cols.append(first)
        if j < K - 1:
            cand = jnp.where(oidx == first, big, cand)
    base = pl.program_id(0) * N
    idx_ref[...] = (jnp.concatenate(cols, axis=1).astype(jnp.int32)
                    + base)


def _sc_gather_body(fdup_hbm, idxg_hbm, out_hbm, idx_v, rows_v, fq_v, gsem):
    wid = lax.axis_index("s") * NC + lax.axis_index("c")
    qbase = wid * QW
    pltpu.sync_copy(idxg_hbm.at[pl.ds(wid * NCH, NCH)], idx_v)
    pltpu.sync_copy(fdup_hbm.at[pl.ds(qbase, QW)], fq_v)
    copies = []
    for c in range(NCH):
        copies.append(
            pltpu.async_copy(
                fdup_hbm.at[idx_v.at[c]],
                rows_v.at[pl.ds(c * GCH, GCH)],
                gsem))
    for cp in copies:
        cp.wait()

    half = lax.iota(jnp.int32, 2 * C) < C

    def body(qg, _):
        for u in range(4):
            qi = qg * 4 + u
            fq = fq_v[qi]
            for j in range(K):
                r = qi * K + j
                v = rows_v[r]
                rows_v[r] = jnp.where(half, v - fq, fq)
        return _

    lax.fori_loop(0, QW // 4, body, None)
    pltpu.sync_copy(rows_v, out_hbm.at[pl.ds(wid * RW, RW)])


@functools.lru_cache(maxsize=1)
def _make_sc_gather():
    return pl.kernel(
        _sc_gather_body,
        out_type=jax.ShapeDtypeStruct((B * N * K, 2 * C), jnp.float32),
        mesh=plsc.VectorSubcoreMesh(
            core_axis_name="c", subcore_axis_name="s"),
        scratch_types=[
            pltpu.VMEM((NCH, GCH), jnp.int32),
            pltpu.VMEM((RW, 2 * C), jnp.float32),
            pltpu.VMEM((QW, 2 * C), jnp.float32),
            pltpu.SemaphoreType.DMA,
        ],
        compiler_params=pltpu.CompilerParams(use_tc_tiling_on_sc=False),
    )


def kernel(points, W, b):
    points_t = jnp.transpose(points, (0, 2, 1))
    bias = b.reshape(1, C)
    grid = (B, N // TQ)
    fdup, idxg = pl.pallas_call(
        _knn_idx_body,
        grid=grid,
        in_specs=[
            pl.BlockSpec((1, TQ, 3), lambda i, j: (i, j, 0)),
            pl.BlockSpec((1, 3, N), lambda i, j: (i, 0, 0)),
            pl.BlockSpec((C, 3), lambda i, j: (0, 0)),
            pl.BlockSpec((1, C), lambda i, j: (0, 0)),
        ],
        out_specs=[
            pl.BlockSpec((TQ, 2 * C), lambda i, j: (i * (N // TQ) + j, 0)),
            pl.BlockSpec((TQ, K), lambda i, j: (i * (N // TQ) + j, 0)),
        ],
        out_shape=[
            jax.ShapeDtypeStruct((B * N, 2 * C), jnp.float32),
            jax.ShapeDtypeStruct((B * N, K), jnp.int32),
        ],
        scratch_shapes=[pltpu.VMEM((1, N), jnp.float32)],
    )(points, points_t, W, bias)
    out = _make_sc_gather()(fdup, idxg.reshape(B * N * K // GCH, GCH))
    return jnp.transpose(out.reshape(B, N, K, 2 * C), (0, 3, 1, 2))


# speedup vs baseline: 51.4381x; 1.0104x over previous
import functools

import jax
import jax.numpy as jnp
from jax import lax
from jax.experimental import pallas as pl
from jax.experimental.pallas import tpu as pltpu
from jax.experimental.pallas import tpu_sc as plsc

B = 4
N = 4096
K = 8
C = 8
TQ = 256

NC = 2
NS = 16
NW = NC * NS
QW = (B * N) // NW
RW = QW * K
GCH = 128
NCH = RW // GCH


def _knn_idx_body(q_ref, kt_ref, w_ref, b_ref, fdup_ref, idx_ref):
    q = q_ref[0]
    kt = kt_ref[0]
    w = w_ref[...]
    bias = b_ref[...]

    sq_k = jnp.sum(kt * kt, axis=0, keepdims=True)
    sq_q = jnp.sum(q * q, axis=1, keepdims=True)
    inner = jax.lax.dot_general(
        q, kt, (((1,), (0,)), ((), ())),
        preferred_element_type=jnp.float32)
    d = sq_q - 2.0 * inner + sq_k

    f_q = jax.lax.dot_general(
        q, w, (((1,), (1,)), ((), ())),
        preferred_element_type=jnp.float32,
        precision=jax.lax.Precision.HIGHEST) + bias
    fdup_ref[...] = jnp.concatenate([f_q, f_q], axis=1)

    keys = (jax.lax.bitcast_convert_type(d + 1.0, jnp.uint32)
            & jnp.uint32(0xFFFFFFE0))
    big = jnp.float32(3.0e38)
    planes = [jnp.full((TQ, 128), big, jnp.float32) for _ in range(4)]
    for c in range(32):
        cv = jax.lax.bitcast_convert_type(
            keys[:, c * 128:(c + 1) * 128] | jnp.uint32(c), jnp.float32)
        for l in range(4):
            lo = jnp.minimum(planes[l], cv)
            cv = jnp.maximum(planes[l], cv)
            planes[l] = lo

    lane = jax.lax.broadcasted_iota(
        jnp.int32, (TQ, 128), 1).astype(jnp.float32)
    cand = jnp.concatenate(planes, axis=1)
    oidx = jnp.concatenate(
        [(jax.lax.bitcast_convert_type(p, jnp.uint32)
          & jnp.uint32(31)).astype(jnp.float32) * 128.0 + lane
         for p in planes], axis=1)
    nf = jnp.float32(N)
    cols = []
    for j in range(K):
        m = jnp.min(cand, axis=1, keepdims=True)
        first = jnp.min(jnp.where(cand == m, oidx, nf), axis=1,
                        keepdims=True)
        cols.append(first)
        if j < K - 1:
            cand = jnp.where(oidx == first, big, cand)
    base = pl.program_id(0) * N
    idx_ref[...] = (jnp.concatenate(cols, axis=1).astype(jnp.int32)
                    + base)


def _sc_gather_body(fdup_hbm, idxg_hbm, out_hbm, idx_v, rows_v, fq_v, gsem):
    wid = lax.axis_index("s") * NC + lax.axis_index("c")
    qbase = wid * QW
    pltpu.sync_copy(idxg_hbm.at[pl.ds(wid * NCH, NCH)], idx_v)
    pltpu.sync_copy(fdup_hbm.at[pl.ds(qbase, QW)], fq_v)
    copies = []
    for c in range(NCH):
        copies.append(
            pltpu.async_copy(
                fdup_hbm.at[idx_v.at[c]],
                rows_v.at[pl.ds(c * GCH, GCH)],
                gsem))
    for cp in copies:
        cp.wait()

    half = lax.iota(jnp.int32, 2 * C) < C

    def body(qi, _):
        fq = fq_v[qi]
        for j in range(K):
            r = qi * K + j
            v = rows_v[r]
            rows_v[r] = jnp.where(half, v - fq, fq)
        return _

    lax.fori_loop(0, QW, body, None)
    pltpu.sync_copy(rows_v, out_hbm.at[pl.ds(wid * RW, RW)])


@functools.lru_cache(maxsize=1)
def _make_sc_gather():
    return pl.kernel(
        _sc_gather_body,
        out_type=jax.ShapeDtypeStruct((B * N * K, 2 * C), jnp.float32),
        mesh=plsc.VectorSubcoreMesh(
            core_axis_name="c", subcore_axis_name="s"),
        scratch_types=[
            pltpu.VMEM((NCH, GCH), jnp.int32),
            pltpu.VMEM((RW, 2 * C), jnp.float32),
            pltpu.VMEM((QW, 2 * C), jnp.float32),
            pltpu.SemaphoreType.DMA,
        ],
        compiler_params=pltpu.CompilerParams(use_tc_tiling_on_sc=False),
    )


def kernel(points, W, b):
    points_t = jnp.transpose(points, (0, 2, 1))
    bias = b.reshape(1, C)
    grid = (B, N // TQ)
    fdup, idxg = pl.pallas_call(
        _knn_idx_body,
        grid=grid,
        in_specs=[
            pl.BlockSpec((1, TQ, 3), lambda i, j: (i, j, 0)),
            pl.BlockSpec((1, 3, N), lambda i, j: (i, 0, 0)),
            pl.BlockSpec((C, 3), lambda i, j: (0, 0)),
            pl.BlockSpec((1, C), lambda i, j: (0, 0)),
        ],
        out_specs=[
            pl.BlockSpec((TQ, 2 * C), lambda i, j: (i * (N // TQ) + j, 0)),
            pl.BlockSpec((TQ, K), lambda i, j: (i * (N // TQ) + j, 0)),
        ],
        out_shape=[
            jax.ShapeDtypeStruct((B * N, 2 * C), jnp.float32),
            jax.ShapeDtypeStruct((B * N, K), jnp.int32),
        ],
    )(points, points_t, W, bias)
    out = _make_sc_gather()(fdup, idxg.reshape(B * N * K // GCH, GCH))
    return jnp.transpose(out.reshape(B, N, K, 2 * C), (0, 3, 1, 2))
